# baseline (speedup 1.0000x reference)
import jax
import jax.numpy as jnp
from jax.experimental import pallas as pl
from jax.experimental.pallas import tpu as pltpu


def _body(x_ref, o_ref, ones_vmem, sem):
    del x_ref
    ones_vmem[...] = jnp.ones_like(ones_vmem)
    cp = pltpu.make_async_copy(
        ones_vmem, o_ref.at[pl.ds(0, ones_vmem.shape[0]), :], sem
    )
    cp.start()
    cp.wait()


def kernel(x):
    n, d = x.shape
    x = jnp.copy(x)
    return pl.pallas_call(
        _body,
        in_specs=[pl.BlockSpec(memory_space=pltpu.MemorySpace.HBM)],
        out_specs=pl.BlockSpec(memory_space=pltpu.MemorySpace.HBM),
        out_shape=jax.ShapeDtypeStruct((n, d), x.dtype),
        input_output_aliases={0: 0},
        scratch_shapes=[
            pltpu.VMEM((2, d), x.dtype),
            pltpu.SemaphoreType.DMA,
        ],
    )(x)
